# baseline (speedup 1.0000x reference)
import functools

import jax
import jax.numpy as jnp
from jax import lax
from jax.experimental import pallas as pl
from jax.experimental.pallas import tpu as pltpu
from jax.experimental.pallas import tpu_sc as plsc

_EPS_NORM = 1e-12
_EPS_COS = 1e-8
_LANES = 16


def _normalize_body(w_ref, out_ref):
    x = w_ref[...]
    s = jnp.sum(x * x, axis=1, keepdims=True)
    out_ref[...] = (x / jnp.maximum(jnp.sqrt(s), _EPS_NORM)).astype(
        jnp.bfloat16)


def _normalize_table(w, rows_per_block=1000):
    v, d = w.shape
    assert v % rows_per_block == 0
    return pl.pallas_call(
        _normalize_body,
        grid=(v // rows_per_block,),
        in_specs=[pl.BlockSpec((rows_per_block, d), lambda i: (i, 0))],
        out_specs=pl.BlockSpec((rows_per_block, d), lambda i: (i, 0)),
        out_shape=jax.ShapeDtypeStruct((v, d), jnp.bfloat16),
    )(w)


def _cos_body(q_ref, d_ref, out_ref):
    q = q_ref[...].astype(jnp.float32)
    d = d_ref[...].astype(jnp.float32)
    dot = jnp.sum(q * d, axis=1)
    nq = jnp.maximum(jnp.sqrt(jnp.sum(q * q, axis=1)), _EPS_COS)
    nd = jnp.maximum(jnp.sqrt(jnp.sum(d * d, axis=1)), _EPS_COS)
    out_ref[...] = dot / (nq * nd)


def _cosine(maxq, maxd, rows_per_block=512):
    b, d = maxq.shape
    assert b % rows_per_block == 0
    return pl.pallas_call(
        _cos_body,
        grid=(b // rows_per_block,),
        in_specs=[pl.BlockSpec((rows_per_block, d), lambda i: (i, 0))] * 2,
        out_specs=pl.BlockSpec((rows_per_block,), lambda i: (i,)),
        out_shape=jax.ShapeDtypeStruct((b,), jnp.float32),
    )(maxq, maxd)


def _make_sc_maxpool(bsz, d, ldp, lqp, n_cores, n_subcores, chunk,
                     interpret=False):
    n_workers = n_cores * n_subcores
    per_w = bsz // n_workers
    assert per_w % chunk == 0
    n_chunks = per_w // chunk
    lt = ldp + lqp
    half = lt // 2
    assert half % 8 == 0 and half <= 128 and ldp % _LANES == 0
    ngd = ldp // _LANES
    ngt = lt // _LANES
    mesh = plsc.VectorSubcoreMesh(
        core_axis_name="c", subcore_axis_name="s",
        num_cores=n_cores, num_subcores=n_subcores)

    lanes2 = 2 * _LANES
    dw = d // 2

    @functools.partial(
        pl.kernel,
        out_type=(
            jax.ShapeDtypeStruct((bsz, dw), jnp.int32),
            jax.ShapeDtypeStruct((bsz, dw), jnp.int32),
        ),
        mesh=mesh,
        interpret=interpret,
        compiler_params=pltpu.CompilerParams(
            use_tc_tiling_on_sc=False, needs_layout_passes=False),
        scratch_types=[
            pltpu.VMEM((chunk, 2, half), jnp.int32),
            pltpu.VMEM((chunk, lt), jnp.float32),
            pltpu.VMEM((lt, dw), jnp.int32),
            pltpu.VMEM((lt, dw), jnp.int32),
            pltpu.VMEM((chunk, dw), jnp.int32),
            pltpu.VMEM((chunk, dw), jnp.int32),
            pltpu.SemaphoreType.DMA,
            pltpu.SemaphoreType.DMA,
        ],
    )
    def sc_kernel(table, idx, mask, outq, outd,
                  idx_v, mask_v, r0, r1, oq_v, od_v, sem0, sem1):
        wid = lax.axis_index("s") * n_cores + lax.axis_index("c")
        w_base = wid * per_w

        def copies(bl, rows, sem):
            return (
                pltpu.make_async_copy(
                    table.at[idx_v.at[bl, 0]], rows.at[pl.ds(0, half)], sem),
                pltpu.make_async_copy(
                    table.at[idx_v.at[bl, 1]], rows.at[pl.ds(half, half)],
                    sem),
            )

        def fire(bl, rows, sem):
            for cp in copies(bl, rows, sem):
                cp.start()

        def wait(bl, rows, sem):
            for cp in copies(bl, rows, sem):
                cp.wait()

        neg = jnp.full((lanes2,), -jnp.inf, jnp.bfloat16)
        nj2 = d // lanes2

        def row_max(bl, rows_ref, g_lo, g_hi):
            def gbody(g, acc):
                mvec = mask_v[bl, pl.ds(g * _LANES, _LANES)]
                for i in range(_LANES):
                    l = g * _LANES + i
                    mb = jnp.broadcast_to(mvec[i], (_LANES,))
                    m = plsc.pack(mb, mb, format=plsc.PackFormat.INTERLEAVED)
                    acc = tuple(
                        jnp.maximum(
                            acc[j],
                            plsc.bitcast(
                                rows_ref[l, pl.ds(j * _LANES, _LANES)],
                                jnp.bfloat16) * m)
                        for j in range(nj2))
                return acc
            return lax.fori_loop(g_lo, g_hi, gbody, (neg,) * nj2)

        def compute(bl, rows):
            accd = row_max(bl, rows, 0, ngd)
            accq = row_max(bl, rows, ngd, ngt)
            for j in range(nj2):
                od_v[bl, pl.ds(j * _LANES, _LANES)] = plsc.bitcast(
                    accd[j], jnp.int32)
                oq_v[bl, pl.ds(j * _LANES, _LANES)] = plsc.bitcast(
                    accq[j], jnp.int32)

        def chunk_body(ci, carry):
            base = w_base + ci * chunk
            pltpu.sync_copy(idx.at[pl.ds(base, chunk)], idx_v)
            pltpu.sync_copy(mask.at[pl.ds(base, chunk)], mask_v)

            fire(0, r0, sem0)

            def step(k, carry):
                b0 = 2 * k
                b1 = b0 + 1
                fire(b1, r1, sem1)
                wait(b0, r0, sem0)
                compute(b0, r0)

                @pl.when(b1 + 1 < chunk)
                def _():
                    fire(b1 + 1, r0, sem0)
                wait(b1, r1, sem1)
                compute(b1, r1)
                return carry

            lax.fori_loop(0, chunk // 2, step, 0)
            pltpu.sync_copy(oq_v, outq.at[pl.ds(base, chunk)])
            pltpu.sync_copy(od_v, outd.at[pl.ds(base, chunk)])
            return carry

        lax.fori_loop(0, n_chunks, chunk_body, 0)

    return sc_kernel


def kernel(inputs_d, inputs_q, mask_d, mask_q, emb_weight):
    bsz, ld = inputs_d.shape
    _, lq = inputs_q.shape
    v, d = emb_weight.shape

    ldp = ld + (-ld) % 16
    pad_d = ldp - ld
    pad_q = (-lq) % 16
    lqp = lq + pad_q
    idx = jnp.concatenate(
        [inputs_d, inputs_d[:, :pad_d], inputs_q, inputs_q[:, :pad_q]],
        axis=1).astype(jnp.int32)
    mask = jnp.concatenate(
        [mask_d, mask_d[:, :pad_d], mask_q, mask_q[:, :pad_q]], axis=1)
    idx = idx.reshape(bsz, 2, (ldp + lqp) // 2)

    table_n = _normalize_table(emb_weight)
    table_p = jax.lax.bitcast_convert_type(
        table_n.reshape(v, d // 2, 2), jnp.int32)

    info = plsc.get_sparse_core_info()
    sc = _make_sc_maxpool(bsz, d, ldp, lqp, info.num_cores,
                          info.num_subcores, chunk=32)
    maxq_p, maxd_p = sc(table_p, idx, mask)

    def unpack(x):
        return jax.lax.bitcast_convert_type(
            x, jnp.bfloat16).reshape(bsz, d)

    return _cosine(unpack(maxq_p), unpack(maxd_p))


# speedup vs baseline: 2.0359x; 2.0359x over previous
import functools

import jax
import jax.numpy as jnp
from jax import lax
from jax.experimental import pallas as pl
from jax.experimental.pallas import tpu as pltpu
from jax.experimental.pallas import tpu_sc as plsc

_EPS_NORM = 1e-12
_EPS_COS = 1e-8
_LANES = 16


def _normalize_body(w_ref, out_ref):
    x = w_ref[...]
    s = jnp.sum(x * x, axis=1, keepdims=True)
    out_ref[...] = (x / jnp.maximum(jnp.sqrt(s), _EPS_NORM)).astype(
        jnp.bfloat16)


def _normalize_table(w, rows_per_block=1000):
    v, d = w.shape
    assert v % rows_per_block == 0
    return pl.pallas_call(
        _normalize_body,
        grid=(v // rows_per_block,),
        in_specs=[pl.BlockSpec((rows_per_block, d), lambda i: (i, 0))],
        out_specs=pl.BlockSpec((rows_per_block, d), lambda i: (i, 0)),
        out_shape=jax.ShapeDtypeStruct((v, d), jnp.bfloat16),
    )(w)


def _cos_body(q_ref, d_ref, out_ref):
    q = q_ref[...].astype(jnp.float32)
    d = d_ref[...].astype(jnp.float32)
    dot = jnp.sum(q * d, axis=1)
    nq = jnp.maximum(jnp.sqrt(jnp.sum(q * q, axis=1)), _EPS_COS)
    nd = jnp.maximum(jnp.sqrt(jnp.sum(d * d, axis=1)), _EPS_COS)
    out_ref[...] = dot / (nq * nd)


def _cosine(maxq, maxd, rows_per_block=512):
    b, d = maxq.shape
    assert b % rows_per_block == 0
    return pl.pallas_call(
        _cos_body,
        grid=(b // rows_per_block,),
        in_specs=[pl.BlockSpec((rows_per_block, d), lambda i: (i, 0))] * 2,
        out_specs=pl.BlockSpec((rows_per_block,), lambda i: (i,)),
        out_shape=jax.ShapeDtypeStruct((b,), jnp.float32),
    )(maxq, maxd)


def _make_sc_maxpool(bsz, d, ldp, lqp, n_cores, n_subcores, chunk,
                     interpret=False):
    n_workers = n_cores * n_subcores
    per_w = bsz // n_workers
    assert per_w % chunk == 0
    n_chunks = per_w // chunk
    lt = ldp + lqp
    half = lt // 2
    assert half % 8 == 0 and half <= 128 and ldp % _LANES == 0
    ngd = ldp // _LANES
    ngt = lt // _LANES
    mesh = plsc.VectorSubcoreMesh(
        core_axis_name="c", subcore_axis_name="s",
        num_cores=n_cores, num_subcores=n_subcores)

    lanes2 = 2 * _LANES
    dw = d // 2

    @functools.partial(
        pl.kernel,
        out_type=(
            jax.ShapeDtypeStruct((bsz, d), jnp.bfloat16),
            jax.ShapeDtypeStruct((bsz, d), jnp.bfloat16),
        ),
        mesh=mesh,
        interpret=interpret,
        compiler_params=pltpu.CompilerParams(
            use_tc_tiling_on_sc=False, needs_layout_passes=False),
        scratch_types=[
            pltpu.VMEM((chunk, 2, half), jnp.int32),
            pltpu.VMEM((chunk, lt), jnp.float32),
            pltpu.VMEM((lt, d), jnp.bfloat16),
            pltpu.VMEM((lt, d), jnp.bfloat16),
            pltpu.VMEM((chunk, d), jnp.bfloat16),
            pltpu.VMEM((chunk, d), jnp.bfloat16),
            pltpu.SemaphoreType.DMA,
            pltpu.SemaphoreType.DMA,
        ],
    )
    def sc_kernel(table, idx, mask, outq, outd,
                  idx_v, mask_v, r0, r1, oq_v, od_v, sem0, sem1):
        wid = lax.axis_index("s") * n_cores + lax.axis_index("c")
        w_base = wid * per_w

        def copies(bl, rows, sem):
            return (
                pltpu.make_async_copy(
                    table.at[idx_v.at[bl, 0]], rows.at[pl.ds(0, half)], sem),
                pltpu.make_async_copy(
                    table.at[idx_v.at[bl, 1]], rows.at[pl.ds(half, half)],
                    sem),
            )

        def fire(bl, rows, sem):
            for cp in copies(bl, rows, sem):
                cp.start()

        def wait(bl, rows, sem):
            for cp in copies(bl, rows, sem):
                cp.wait()

        neg = jnp.full((lanes2,), -jnp.inf, jnp.bfloat16)
        nj2 = d // lanes2

        def row_max(bl, rows_ref, g_lo, g_hi):
            def gbody(g, acc):
                mvec = mask_v[bl, pl.ds(g * _LANES, _LANES)]
                for i in range(_LANES):
                    l = g * _LANES + i
                    mb = jnp.broadcast_to(mvec[i], (_LANES,))
                    m = plsc.pack(mb, mb, format=plsc.PackFormat.INTERLEAVED)
                    acc = tuple(
                        jnp.maximum(
                            acc[j],
                            rows_ref[l, pl.ds(j * lanes2, lanes2)] * m)
                        for j in range(nj2))
                return acc
            return lax.fori_loop(g_lo, g_hi, gbody, (neg,) * nj2)

        def compute(bl, rows):
            accd = row_max(bl, rows, 0, ngd)
            accq = row_max(bl, rows, ngd, ngt)
            for j in range(nj2):
                od_v[bl, pl.ds(j * lanes2, lanes2)] = accd[j]
                oq_v[bl, pl.ds(j * lanes2, lanes2)] = accq[j]

        def chunk_body(ci, carry):
            base = w_base + ci * chunk
            pltpu.sync_copy(idx.at[pl.ds(base, chunk)], idx_v)
            pltpu.sync_copy(mask.at[pl.ds(base, chunk)], mask_v)

            fire(0, r0, sem0)

            def step(k, carry):
                b0 = 2 * k
                b1 = b0 + 1
                fire(b1, r1, sem1)
                wait(b0, r0, sem0)
                compute(b0, r0)

                @pl.when(b1 + 1 < chunk)
                def _():
                    fire(b1 + 1, r0, sem0)
                wait(b1, r1, sem1)
                compute(b1, r1)
                return carry

            lax.fori_loop(0, chunk // 2, step, 0)
            pltpu.sync_copy(oq_v, outq.at[pl.ds(base, chunk)])
            pltpu.sync_copy(od_v, outd.at[pl.ds(base, chunk)])
            return carry

        lax.fori_loop(0, n_chunks, chunk_body, 0)

    return sc_kernel


def kernel(inputs_d, inputs_q, mask_d, mask_q, emb_weight):
    bsz, ld = inputs_d.shape
    _, lq = inputs_q.shape
    v, d = emb_weight.shape

    ldp = ld + (-ld) % 16
    pad_d = ldp - ld
    pad_q = (-lq) % 16
    lqp = lq + pad_q
    idx = jnp.concatenate(
        [inputs_d, inputs_d[:, :pad_d], inputs_q, inputs_q[:, :pad_q]],
        axis=1).astype(jnp.int32)
    mask = jnp.concatenate(
        [mask_d, mask_d[:, :pad_d], mask_q, mask_q[:, :pad_q]], axis=1)
    idx = idx.reshape(bsz, 2, (ldp + lqp) // 2)

    table_n = _normalize_table(emb_weight)

    info = plsc.get_sparse_core_info()
    sc = _make_sc_maxpool(bsz, d, ldp, lqp, info.num_cores,
                          info.num_subcores, chunk=32)
    maxq, maxd = sc(table_n, idx, mask)
    return _cosine(maxq, maxd)


# speedup vs baseline: 2.1565x; 1.0592x over previous
import functools

import jax
import jax.numpy as jnp
from jax import lax
from jax.experimental import pallas as pl
from jax.experimental.pallas import tpu as pltpu
from jax.experimental.pallas import tpu_sc as plsc

_EPS_NORM = 1e-12
_EPS_COS = 1e-8
_LANES = 16


def _normalize_body(w_ref, out_ref):
    x = w_ref[...]
    s = jnp.sum(x * x, axis=1, keepdims=True)
    out_ref[...] = (x * lax.rsqrt(jnp.maximum(s, _EPS_NORM * _EPS_NORM))
                    ).astype(jnp.bfloat16)


def _normalize_table(w, rows_per_block=1000):
    v, d = w.shape
    assert v % rows_per_block == 0
    return pl.pallas_call(
        _normalize_body,
        grid=(v // rows_per_block,),
        in_specs=[pl.BlockSpec((rows_per_block, d), lambda i: (i, 0))],
        out_specs=pl.BlockSpec((rows_per_block, d), lambda i: (i, 0)),
        out_shape=jax.ShapeDtypeStruct((v, d), jnp.bfloat16),
    )(w)


def _cos_body(q_ref, d_ref, out_ref):
    q = q_ref[...].astype(jnp.float32)
    d = d_ref[...].astype(jnp.float32)
    dot = jnp.sum(q * d, axis=1)
    nq = jnp.maximum(jnp.sqrt(jnp.sum(q * q, axis=1)), _EPS_COS)
    nd = jnp.maximum(jnp.sqrt(jnp.sum(d * d, axis=1)), _EPS_COS)
    out_ref[...] = dot / (nq * nd)


def _cosine(maxq, maxd, rows_per_block=512):
    b, d = maxq.shape
    assert b % rows_per_block == 0
    return pl.pallas_call(
        _cos_body,
        grid=(b // rows_per_block,),
        in_specs=[pl.BlockSpec((rows_per_block, d), lambda i: (i, 0))] * 2,
        out_specs=pl.BlockSpec((rows_per_block,), lambda i: (i,)),
        out_shape=jax.ShapeDtypeStruct((b,), jnp.float32),
    )(maxq, maxd)


def _make_sc_maxpool(bsz, d, ldp, lqp, n_cores, n_subcores, chunk,
                     interpret=False):
    n_workers = n_cores * n_subcores
    per_w = bsz // n_workers
    assert per_w % chunk == 0
    n_chunks = per_w // chunk
    lt = ldp + lqp
    half = lt // 2
    assert half % 8 == 0 and half <= 128 and ldp % _LANES == 0
    ngd = ldp // _LANES
    ngt = lt // _LANES
    mesh = plsc.VectorSubcoreMesh(
        core_axis_name="c", subcore_axis_name="s",
        num_cores=n_cores, num_subcores=n_subcores)

    lanes2 = 2 * _LANES
    dw = d // 2

    @functools.partial(
        pl.kernel,
        out_type=(
            jax.ShapeDtypeStruct((bsz, d), jnp.bfloat16),
            jax.ShapeDtypeStruct((bsz, d), jnp.bfloat16),
        ),
        mesh=mesh,
        interpret=interpret,
        compiler_params=pltpu.CompilerParams(
            use_tc_tiling_on_sc=False, needs_layout_passes=False),
        scratch_types=[
            pltpu.VMEM((chunk * lt,), jnp.int32),
            pltpu.VMEM((chunk * lt,), jnp.float32),
            pltpu.VMEM((lt, d), jnp.bfloat16),
            pltpu.VMEM((lt, d), jnp.bfloat16),
            pltpu.VMEM((chunk, d), jnp.bfloat16),
            pltpu.VMEM((chunk, d), jnp.bfloat16),
            pltpu.SemaphoreType.DMA,
            pltpu.SemaphoreType.DMA,
        ],
    )
    def sc_kernel(table, idx, mask, outq, outd,
                  idx_v, mask_v, r0, r1, oq_v, od_v, sem0, sem1):
        wid = lax.axis_index("s") * n_cores + lax.axis_index("c")
        w_base = wid * per_w

        def copies(bl, rows, sem):
            return (
                pltpu.make_async_copy(
                    table.at[idx_v.at[pl.ds(bl * lt, half)]],
                    rows.at[pl.ds(0, half)], sem),
                pltpu.make_async_copy(
                    table.at[idx_v.at[pl.ds(bl * lt + half, half)]],
                    rows.at[pl.ds(half, half)], sem),
            )

        def fire(bl, rows, sem):
            for cp in copies(bl, rows, sem):
                cp.start()

        def wait(bl, rows, sem):
            for cp in copies(bl, rows, sem):
                cp.wait()

        neg = jnp.full((lanes2,), -jnp.inf, jnp.bfloat16)
        nj2 = d // lanes2

        def row_max(bl, rows_ref, g_lo, g_hi):
            def gbody(g, acc):
                mvec = mask_v[pl.ds(bl * lt + g * _LANES, _LANES)]
                for i in range(_LANES):
                    l = g * _LANES + i
                    mb = jnp.broadcast_to(mvec[i], (_LANES,))
                    m = plsc.pack(mb, mb, format=plsc.PackFormat.INTERLEAVED)
                    acc = tuple(
                        jnp.maximum(
                            acc[j],
                            rows_ref[l, pl.ds(j * lanes2, lanes2)] * m)
                        for j in range(nj2))
                return acc
            return lax.fori_loop(g_lo, g_hi, gbody, (neg,) * nj2)

        def compute(bl, rows):
            accd = row_max(bl, rows, 0, ngd)
            accq = row_max(bl, rows, ngd, ngt)
            for j in range(nj2):
                od_v[bl, pl.ds(j * lanes2, lanes2)] = accd[j]
                oq_v[bl, pl.ds(j * lanes2, lanes2)] = accq[j]

        def chunk_body(ci, carry):
            base = w_base + ci * chunk
            pltpu.sync_copy(idx.at[pl.ds(base * lt, chunk * lt)], idx_v)
            pltpu.sync_copy(mask.at[pl.ds(base * lt, chunk * lt)], mask_v)

            fire(0, r0, sem0)

            def step(k, carry):
                b0 = 2 * k
                b1 = b0 + 1
                fire(b1, r1, sem1)
                wait(b0, r0, sem0)
                compute(b0, r0)

                @pl.when(b1 + 1 < chunk)
                def _():
                    fire(b1 + 1, r0, sem0)
                wait(b1, r1, sem1)
                compute(b1, r1)
                return carry

            lax.fori_loop(0, chunk // 2, step, 0)
            pltpu.sync_copy(oq_v, outq.at[pl.ds(base, chunk)])
            pltpu.sync_copy(od_v, outd.at[pl.ds(base, chunk)])
            return carry

        lax.fori_loop(0, n_chunks, chunk_body, 0)

    return sc_kernel


def kernel(inputs_d, inputs_q, mask_d, mask_q, emb_weight):
    bsz, ld = inputs_d.shape
    _, lq = inputs_q.shape
    v, d = emb_weight.shape

    ldp = ld + (-ld) % 16
    pad_d = ldp - ld
    pad_q = (-lq) % 16
    lqp = lq + pad_q
    idx = jnp.concatenate(
        [inputs_d, inputs_d[:, :pad_d], inputs_q, inputs_q[:, :pad_q]],
        axis=1).astype(jnp.int32).reshape(-1)
    mask = jnp.concatenate(
        [mask_d, mask_d[:, :pad_d], mask_q, mask_q[:, :pad_q]],
        axis=1).reshape(-1)

    table_n = _normalize_table(emb_weight)

    info = plsc.get_sparse_core_info()
    sc = _make_sc_maxpool(bsz, d, ldp, lqp, info.num_cores,
                          info.num_subcores, chunk=32)
    maxq, maxd = sc(table_n, idx, mask)
    return _cosine(maxq, maxd)


# speedup vs baseline: 2.2189x; 1.0289x over previous
import functools

import jax
import jax.numpy as jnp
from jax import lax
from jax.experimental import pallas as pl
from jax.experimental.pallas import tpu as pltpu
from jax.experimental.pallas import tpu_sc as plsc

_EPS_NORM = 1e-12
_EPS_COS = 1e-8
_LANES = 16


def _normalize_body(w_ref, out_ref):
    x = w_ref[...]
    s = jnp.sum(x * x, axis=1, keepdims=True)
    out_ref[...] = (x * lax.rsqrt(jnp.maximum(s, _EPS_NORM * _EPS_NORM))
                    ).astype(jnp.bfloat16)


def _normalize_table(w, rows_per_block=1000):
    v, d = w.shape
    assert v % rows_per_block == 0
    return pl.pallas_call(
        _normalize_body,
        grid=(v // rows_per_block,),
        in_specs=[pl.BlockSpec((rows_per_block, d), lambda i: (i, 0))],
        out_specs=pl.BlockSpec((rows_per_block, d), lambda i: (i, 0)),
        out_shape=jax.ShapeDtypeStruct((v, d), jnp.bfloat16),
    )(w)


def _cos_body(q_ref, d_ref, out_ref):
    q = q_ref[...].astype(jnp.float32)
    d = d_ref[...].astype(jnp.float32)
    dot = jnp.sum(q * d, axis=1)
    nq = jnp.maximum(jnp.sqrt(jnp.sum(q * q, axis=1)), _EPS_COS)
    nd = jnp.maximum(jnp.sqrt(jnp.sum(d * d, axis=1)), _EPS_COS)
    out_ref[...] = dot / (nq * nd)


def _cosine(maxq, maxd, rows_per_block=512):
    b, d = maxq.shape
    assert b % rows_per_block == 0
    return pl.pallas_call(
        _cos_body,
        grid=(b // rows_per_block,),
        in_specs=[pl.BlockSpec((rows_per_block, d), lambda i: (i, 0))] * 2,
        out_specs=pl.BlockSpec((rows_per_block,), lambda i: (i,)),
        out_shape=jax.ShapeDtypeStruct((b,), jnp.float32),
    )(maxq, maxd)


def _make_sc_maxpool(bsz, d, ldp, lqp, n_cores, n_subcores, chunk,
                     interpret=False):
    n_workers = n_cores * n_subcores
    per_w = bsz // n_workers
    assert per_w % chunk == 0
    n_chunks = per_w // chunk
    lt = ldp + lqp
    half = lt // 2
    assert half % 8 == 0 and half <= 128 and ldp % _LANES == 0
    ngd = ldp // _LANES
    ngt = lt // _LANES
    mesh = plsc.VectorSubcoreMesh(
        core_axis_name="c", subcore_axis_name="s",
        num_cores=n_cores, num_subcores=n_subcores)

    lanes2 = 2 * _LANES
    dw = d // 2

    @functools.partial(
        pl.kernel,
        out_type=(
            jax.ShapeDtypeStruct((bsz, d), jnp.bfloat16),
            jax.ShapeDtypeStruct((bsz, d), jnp.bfloat16),
        ),
        mesh=mesh,
        interpret=interpret,
        compiler_params=pltpu.CompilerParams(
            use_tc_tiling_on_sc=False, needs_layout_passes=False),
        scratch_types=[
            pltpu.VMEM((chunk * lt,), jnp.int32),
            pltpu.VMEM((chunk * lt,), jnp.int32),
            pltpu.VMEM((lt, d), jnp.bfloat16),
            pltpu.VMEM((lt, d), jnp.bfloat16),
            pltpu.VMEM((chunk, d), jnp.bfloat16),
            pltpu.VMEM((chunk, d), jnp.bfloat16),
            pltpu.SemaphoreType.DMA,
            pltpu.SemaphoreType.DMA,
        ],
    )
    def sc_kernel(table, wm, outq, outd,
                  wm_v, idx_v, r0, r1, oq_v, od_v, sem0, sem1):
        wid = lax.axis_index("s") * n_cores + lax.axis_index("c")
        w_base = wid * per_w

        def copies(bl, rows, sem):
            return (
                pltpu.make_async_copy(
                    table.at[idx_v.at[pl.ds(bl * lt, half)]],
                    rows.at[pl.ds(0, half)], sem),
                pltpu.make_async_copy(
                    table.at[idx_v.at[pl.ds(bl * lt + half, half)]],
                    rows.at[pl.ds(half, half)], sem),
            )

        def fire(bl, rows, sem):
            for cp in copies(bl, rows, sem):
                cp.start()

        def wait(bl, rows, sem):
            for cp in copies(bl, rows, sem):
                cp.wait()

        neg = jnp.full((lanes2,), -jnp.inf, jnp.bfloat16)
        nj2 = d // lanes2

        def row_max(bl, rows_ref, g_lo, g_hi):
            def gbody(g, acc):
                wvec = wm_v[pl.ds(bl * lt + g * _LANES, _LANES)]
                mvec = lax.shift_right_logical(wvec, 17).astype(
                    jnp.float32) * (1.0 / 32767.0)
                for i in range(_LANES):
                    l = g * _LANES + i
                    mb = jnp.broadcast_to(mvec[i], (_LANES,))
                    m = plsc.pack(mb, mb, format=plsc.PackFormat.INTERLEAVED)
                    acc = tuple(
                        jnp.maximum(
                            acc[j],
                            rows_ref[l, pl.ds(j * lanes2, lanes2)] * m)
                        for j in range(nj2))
                return acc
            return lax.fori_loop(g_lo, g_hi, gbody, (neg,) * nj2)

        def compute(bl, rows):
            accd = row_max(bl, rows, 0, ngd)
            accq = row_max(bl, rows, ngd, ngt)
            for j in range(nj2):
                od_v[bl, pl.ds(j * lanes2, lanes2)] = accd[j]
                oq_v[bl, pl.ds(j * lanes2, lanes2)] = accq[j]

        def chunk_body(ci, carry):
            base = w_base + ci * chunk
            pltpu.sync_copy(wm.at[pl.ds(base * lt, chunk * lt)], wm_v)

            def decode(t, carry):
                w16 = wm_v[pl.ds(t * _LANES, _LANES)]
                idx_v[pl.ds(t * _LANES, _LANES)] = w16 & 0x1FFFF
                return carry
            lax.fori_loop(0, chunk * lt // _LANES, decode, 0)

            fire(0, r0, sem0)

            def step(k, carry):
                b0 = 2 * k
                b1 = b0 + 1
                fire(b1, r1, sem1)
                wait(b0, r0, sem0)
                compute(b0, r0)

                @pl.when(b1 + 1 < chunk)
                def _():
                    fire(b1 + 1, r0, sem0)
                wait(b1, r1, sem1)
                compute(b1, r1)
                return carry

            lax.fori_loop(0, chunk // 2, step, 0)
            pltpu.sync_copy(oq_v, outq.at[pl.ds(base, chunk)])
            pltpu.sync_copy(od_v, outd.at[pl.ds(base, chunk)])
            return carry

        lax.fori_loop(0, n_chunks, chunk_body, 0)

    return sc_kernel


def kernel(inputs_d, inputs_q, mask_d, mask_q, emb_weight):
    bsz, ld = inputs_d.shape
    _, lq = inputs_q.shape
    v, d = emb_weight.shape

    ldp = ld + (-ld) % 16
    pad_d = ldp - ld
    pad_q = (-lq) % 16
    lqp = lq + pad_q
    idx = jnp.concatenate(
        [inputs_d, inputs_d[:, :pad_d], inputs_q, inputs_q[:, :pad_q]],
        axis=1).astype(jnp.int32)
    mask = jnp.concatenate(
        [mask_d, mask_d[:, :pad_d], mask_q, mask_q[:, :pad_q]], axis=1)
    m15 = jnp.round(mask * 32767.0).astype(jnp.int32)
    wm = (idx | (m15 << 17)).reshape(-1)

    table_n = _normalize_table(emb_weight)

    info = plsc.get_sparse_core_info()
    sc = _make_sc_maxpool(bsz, d, ldp, lqp, info.num_cores,
                          info.num_subcores, chunk=32)
    maxq, maxd = sc(table_n, wm)
    return _cosine(maxq, maxd)


# speedup vs baseline: 2.4368x; 1.0982x over previous
import functools

import jax
import jax.numpy as jnp
from jax import lax
from jax.experimental import pallas as pl
from jax.experimental.pallas import tpu as pltpu
from jax.experimental.pallas import tpu_sc as plsc

_EPS_NORM = 1e-12
_EPS_COS = 1e-8
_LANES = 16


def _normalize_body(w_ref, out_ref):
    x = w_ref[...]
    s = jnp.sum(x * x, axis=1, keepdims=True)
    out_ref[...] = (x * lax.rsqrt(jnp.maximum(s, _EPS_NORM * _EPS_NORM))
                    ).astype(jnp.bfloat16)


def _normalize_table(w, rows_per_block=1000):
    v, d = w.shape
    assert v % rows_per_block == 0
    return pl.pallas_call(
        _normalize_body,
        grid=(v // rows_per_block,),
        in_specs=[pl.BlockSpec((rows_per_block, d), lambda i: (i, 0))],
        out_specs=pl.BlockSpec((rows_per_block, d), lambda i: (i, 0)),
        out_shape=jax.ShapeDtypeStruct((v, d), jnp.bfloat16),
    )(w)


def _cos_body(q_ref, d_ref, out_ref):
    q = q_ref[...].astype(jnp.float32)
    d = d_ref[...].astype(jnp.float32)
    dot = jnp.sum(q * d, axis=1)
    nq = jnp.maximum(jnp.sqrt(jnp.sum(q * q, axis=1)), _EPS_COS)
    nd = jnp.maximum(jnp.sqrt(jnp.sum(d * d, axis=1)), _EPS_COS)
    out_ref[...] = dot / (nq * nd)


def _cosine(maxq, maxd, rows_per_block=512):
    b, d = maxq.shape
    assert b % rows_per_block == 0
    return pl.pallas_call(
        _cos_body,
        grid=(b // rows_per_block,),
        in_specs=[pl.BlockSpec((rows_per_block, d), lambda i: (i, 0))] * 2,
        out_specs=pl.BlockSpec((rows_per_block,), lambda i: (i,)),
        out_shape=jax.ShapeDtypeStruct((b,), jnp.float32),
    )(maxq, maxd)


def _make_sc_maxpool(bsz, d, ldp, lqp, n_cores, n_subcores, chunk,
                     interpret=False):
    n_workers = n_cores * n_subcores
    per_w = bsz // n_workers
    assert per_w % chunk == 0
    n_chunks = per_w // chunk
    lt = ldp + lqp
    half = lt // 2
    assert half % 8 == 0 and half <= 128 and ldp % _LANES == 0
    ngd = ldp // _LANES
    ngt = lt // _LANES
    mesh = plsc.VectorSubcoreMesh(
        core_axis_name="c", subcore_axis_name="s",
        num_cores=n_cores, num_subcores=n_subcores)

    lanes2 = 2 * _LANES
    dw = d // 2

    @functools.partial(
        pl.kernel,
        out_type=(
            jax.ShapeDtypeStruct((bsz, d), jnp.bfloat16),
            jax.ShapeDtypeStruct((bsz, d), jnp.bfloat16),
        ),
        mesh=mesh,
        interpret=interpret,
        compiler_params=pltpu.CompilerParams(
            use_tc_tiling_on_sc=False, needs_layout_passes=False),
        scratch_types=[
            pltpu.VMEM((chunk * lt,), jnp.int32),
            pltpu.VMEM((chunk * lt,), jnp.int32),
            pltpu.VMEM((lt, d), jnp.bfloat16),
            pltpu.VMEM((lt, d), jnp.bfloat16),
            pltpu.VMEM((lt, d), jnp.bfloat16),
            pltpu.VMEM((lt, d), jnp.bfloat16),
            pltpu.VMEM((chunk, d), jnp.bfloat16),
            pltpu.VMEM((chunk, d), jnp.bfloat16),
            pltpu.SemaphoreType.DMA,
            pltpu.SemaphoreType.DMA,
            pltpu.SemaphoreType.DMA,
            pltpu.SemaphoreType.DMA,
        ],
    )
    def sc_kernel(table, wm, outq, outd,
                  wm_v, idx_v, r0, r1, r2, r3, oq_v, od_v,
                  sem0, sem1, sem2, sem3):
        wid = lax.axis_index("s") * n_cores + lax.axis_index("c")
        w_base = wid * per_w

        def copies(bl, rows, sem):
            return (
                pltpu.make_async_copy(
                    table.at[idx_v.at[pl.ds(bl * lt, half)]],
                    rows.at[pl.ds(0, half)], sem),
                pltpu.make_async_copy(
                    table.at[idx_v.at[pl.ds(bl * lt + half, half)]],
                    rows.at[pl.ds(half, half)], sem),
            )

        def fire(bl, rows, sem):
            for cp in copies(bl, rows, sem):
                cp.start()

        def wait(bl, rows, sem):
            for cp in copies(bl, rows, sem):
                cp.wait()

        neg = jnp.full((lanes2,), -jnp.inf, jnp.bfloat16)
        nj2 = d // lanes2

        def row_max(bl, rows_ref, g_lo, g_hi):
            def gbody(g, acc):
                wvec = wm_v[pl.ds(bl * lt + g * _LANES, _LANES)]
                mvec = lax.shift_right_logical(wvec, 17).astype(
                    jnp.float32) * (1.0 / 32767.0)
                for i in range(_LANES):
                    l = g * _LANES + i
                    mb = jnp.broadcast_to(mvec[i], (_LANES,))
                    m = plsc.pack(mb, mb, format=plsc.PackFormat.INTERLEAVED)
                    acc = tuple(
                        jnp.maximum(
                            acc[j],
                            rows_ref[l, pl.ds(j * lanes2, lanes2)] * m)
                        for j in range(nj2))
                return acc
            return lax.fori_loop(g_lo, g_hi, gbody, (neg,) * nj2)

        def compute(bl, rows):
            accd = row_max(bl, rows, 0, ngd)
            accq = row_max(bl, rows, ngd, ngt)
            for j in range(nj2):
                od_v[bl, pl.ds(j * lanes2, lanes2)] = accd[j]
                oq_v[bl, pl.ds(j * lanes2, lanes2)] = accq[j]

        def chunk_body(ci, carry):
            base = w_base + ci * chunk
            pltpu.sync_copy(wm.at[pl.ds(base * lt, chunk * lt)], wm_v)

            def decode(t, carry):
                w16 = wm_v[pl.ds(t * _LANES, _LANES)]
                idx_v[pl.ds(t * _LANES, _LANES)] = w16 & 0x1FFFF
                return carry
            lax.fori_loop(0, chunk * lt // _LANES, decode, 0)

            bufs = ((r0, sem0), (r1, sem1), (r2, sem2), (r3, sem3))
            fire(0, r0, sem0)
            fire(1, r1, sem1)

            def step(k, carry):
                for j in range(4):
                    b = 4 * k + j
                    rows, sem = bufs[j]
                    rows2, sem2_ = bufs[(j + 2) % 4]

                    @pl.when(b + 2 < chunk)
                    def _():
                        fire(b + 2, rows2, sem2_)
                    wait(b, rows, sem)
                    compute(b, rows)
                return carry

            lax.fori_loop(0, chunk // 4, step, 0)
            pltpu.sync_copy(oq_v, outq.at[pl.ds(base, chunk)])
            pltpu.sync_copy(od_v, outd.at[pl.ds(base, chunk)])
            return carry

        lax.fori_loop(0, n_chunks, chunk_body, 0)

    return sc_kernel


def kernel(inputs_d, inputs_q, mask_d, mask_q, emb_weight):
    bsz, ld = inputs_d.shape
    _, lq = inputs_q.shape
    v, d = emb_weight.shape

    ldp = ld + (-ld) % 16
    pad_d = ldp - ld
    pad_q = (-lq) % 16
    lqp = lq + pad_q
    idx = jnp.concatenate(
        [inputs_d, inputs_d[:, :pad_d], inputs_q, inputs_q[:, :pad_q]],
        axis=1).astype(jnp.int32)
    mask = jnp.concatenate(
        [mask_d, mask_d[:, :pad_d], mask_q, mask_q[:, :pad_q]], axis=1)
    m15 = jnp.round(mask * 32767.0).astype(jnp.int32)
    wm = (idx | (m15 << 17)).reshape(-1)

    table_n = _normalize_table(emb_weight)

    info = plsc.get_sparse_core_info()
    sc = _make_sc_maxpool(bsz, d, ldp, lqp, info.num_cores,
                          info.num_subcores, chunk=32)
    maxq, maxd = sc(table_n, wm)
    return _cosine(maxq, maxd)


# speedup vs baseline: 2.4427x; 1.0024x over previous
import functools

import jax
import jax.numpy as jnp
from jax import lax
from jax.experimental import pallas as pl
from jax.experimental.pallas import tpu as pltpu
from jax.experimental.pallas import tpu_sc as plsc

_EPS_NORM = 1e-12
_EPS_COS = 1e-8
_LANES = 16


def _normalize_body(w_ref, out_ref):
    x = w_ref[...]
    s = jnp.sum(x * x, axis=1, keepdims=True)
    out_ref[...] = (x * lax.rsqrt(jnp.maximum(s, _EPS_NORM * _EPS_NORM))
                    ).astype(jnp.bfloat16)


def _normalize_table(w, rows_per_block=1000):
    v, d = w.shape
    assert v % rows_per_block == 0
    return pl.pallas_call(
        _normalize_body,
        grid=(v // rows_per_block,),
        in_specs=[pl.BlockSpec((rows_per_block, d), lambda i: (i, 0))],
        out_specs=pl.BlockSpec((rows_per_block, d), lambda i: (i, 0)),
        out_shape=jax.ShapeDtypeStruct((v, d), jnp.bfloat16),
    )(w)


def _cos_body(q_ref, d_ref, out_ref):
    q = q_ref[...].astype(jnp.float32)
    d = d_ref[...].astype(jnp.float32)
    dot = jnp.sum(q * d, axis=1)
    nq = jnp.maximum(jnp.sqrt(jnp.sum(q * q, axis=1)), _EPS_COS)
    nd = jnp.maximum(jnp.sqrt(jnp.sum(d * d, axis=1)), _EPS_COS)
    out_ref[...] = dot / (nq * nd)


def _cosine(maxq, maxd, rows_per_block=512):
    b, d = maxq.shape
    assert b % rows_per_block == 0
    return pl.pallas_call(
        _cos_body,
        grid=(b // rows_per_block,),
        in_specs=[pl.BlockSpec((rows_per_block, d), lambda i: (i, 0))] * 2,
        out_specs=pl.BlockSpec((rows_per_block,), lambda i: (i,)),
        out_shape=jax.ShapeDtypeStruct((b,), jnp.float32),
    )(maxq, maxd)


def _make_sc_maxpool(bsz, d, ldp, lqp, n_cores, n_subcores, chunk,
                     interpret=False):
    n_workers = n_cores * n_subcores
    per_w = bsz // n_workers
    assert per_w % chunk == 0
    n_chunks = per_w // chunk
    lt = ldp + lqp
    half = lt // 2
    assert half % 8 == 0 and half <= 128 and ldp % _LANES == 0
    ngd = ldp // _LANES
    ngt = lt // _LANES
    mesh = plsc.VectorSubcoreMesh(
        core_axis_name="c", subcore_axis_name="s",
        num_cores=n_cores, num_subcores=n_subcores)

    lanes2 = 2 * _LANES
    dw = d // 2

    @functools.partial(
        pl.kernel,
        out_type=(
            jax.ShapeDtypeStruct((bsz, d), jnp.bfloat16),
            jax.ShapeDtypeStruct((bsz, d), jnp.bfloat16),
        ),
        mesh=mesh,
        interpret=interpret,
        compiler_params=pltpu.CompilerParams(
            use_tc_tiling_on_sc=False, needs_layout_passes=False),
        scratch_types=[
            pltpu.VMEM((chunk * lt,), jnp.int32),
            pltpu.VMEM((chunk * lt,), jnp.int32),
            pltpu.VMEM((lt, d), jnp.bfloat16),
            pltpu.VMEM((lt, d), jnp.bfloat16),
            pltpu.VMEM((lt, d), jnp.bfloat16),
            pltpu.VMEM((lt, d), jnp.bfloat16),
            pltpu.VMEM((chunk, d), jnp.bfloat16),
            pltpu.VMEM((chunk, d), jnp.bfloat16),
            pltpu.SemaphoreType.DMA,
            pltpu.SemaphoreType.DMA,
            pltpu.SemaphoreType.DMA,
            pltpu.SemaphoreType.DMA,
        ],
    )
    def sc_kernel(table, wm, outq, outd,
                  wm_v, idx_v, r0, r1, r2, r3, oq_v, od_v,
                  sem0, sem1, sem2, sem3):
        wid = lax.axis_index("s") * n_cores + lax.axis_index("c")
        w_base = wid * per_w

        def copies(bl, rows, sem):
            return (
                pltpu.make_async_copy(
                    table.at[idx_v.at[pl.ds(bl * lt, half)]],
                    rows.at[pl.ds(0, half)], sem),
                pltpu.make_async_copy(
                    table.at[idx_v.at[pl.ds(bl * lt + half, half)]],
                    rows.at[pl.ds(half, half)], sem),
            )

        def fire(bl, rows, sem):
            for cp in copies(bl, rows, sem):
                cp.start()

        def wait(bl, rows, sem):
            for cp in copies(bl, rows, sem):
                cp.wait()

        neg = jnp.full((lanes2,), -jnp.inf, jnp.bfloat16)
        nj2 = d // lanes2

        def row_max(bl, rows_ref, g_lo, g_hi):
            def gbody(g, acc):
                wvec = wm_v[pl.ds(bl * lt + g * _LANES, _LANES)]
                mvec = lax.shift_right_logical(wvec, 17).astype(
                    jnp.float32) * (1.0 / 32767.0)
                for i in range(_LANES):
                    l = g * _LANES + i
                    mb = jnp.broadcast_to(mvec[i], (_LANES,))
                    m = plsc.pack(mb, mb, format=plsc.PackFormat.INTERLEAVED)
                    acc = tuple(
                        jnp.maximum(
                            acc[j],
                            rows_ref[l, pl.ds(j * lanes2, lanes2)] * m)
                        for j in range(nj2))
                return acc
            return lax.fori_loop(g_lo, g_hi, gbody, (neg,) * nj2)

        def compute(bl, rows):
            accd = row_max(bl, rows, 0, ngd)
            accq = row_max(bl, rows, ngd, ngt)
            for j in range(nj2):
                od_v[bl, pl.ds(j * lanes2, lanes2)] = accd[j]
                oq_v[bl, pl.ds(j * lanes2, lanes2)] = accq[j]

        def chunk_body(ci, carry):
            base = w_base + ci * chunk
            pltpu.sync_copy(wm.at[pl.ds(base * lt, chunk * lt)], wm_v)

            def decode(t, carry):
                w16 = wm_v[pl.ds(t * _LANES, _LANES)]
                idx_v[pl.ds(t * _LANES, _LANES)] = w16 & 0x1FFFF
                return carry
            lax.fori_loop(0, chunk * lt // _LANES, decode, 0)

            bufs = ((r0, sem0), (r1, sem1), (r2, sem2), (r3, sem3))
            fire(0, r0, sem0)
            fire(1, r1, sem1)
            fire(2, r2, sem2)

            def step(k, carry):
                for j in range(4):
                    b = 4 * k + j
                    rows, sem = bufs[j]
                    rows3, sem3_ = bufs[(j + 3) % 4]

                    @pl.when(b + 3 < chunk)
                    def _():
                        fire(b + 3, rows3, sem3_)
                    wait(b, rows, sem)
                    compute(b, rows)
                return carry

            lax.fori_loop(0, chunk // 4, step, 0)
            pltpu.sync_copy(oq_v, outq.at[pl.ds(base, chunk)])
            pltpu.sync_copy(od_v, outd.at[pl.ds(base, chunk)])
            return carry

        lax.fori_loop(0, n_chunks, chunk_body, 0)

    return sc_kernel


def kernel(inputs_d, inputs_q, mask_d, mask_q, emb_weight):
    bsz, ld = inputs_d.shape
    _, lq = inputs_q.shape
    v, d = emb_weight.shape

    ldp = ld + (-ld) % 16
    pad_d = ldp - ld
    pad_q = (-lq) % 16
    lqp = lq + pad_q
    idx = jnp.concatenate(
        [inputs_d, inputs_d[:, :pad_d], inputs_q, inputs_q[:, :pad_q]],
        axis=1).astype(jnp.int32)
    mask = jnp.concatenate(
        [mask_d, mask_d[:, :pad_d], mask_q, mask_q[:, :pad_q]], axis=1)
    m15 = jnp.round(mask * 32767.0).astype(jnp.int32)
    wm = (idx | (m15 << 17)).reshape(-1)

    table_n = _normalize_table(emb_weight)

    info = plsc.get_sparse_core_info()
    sc = _make_sc_maxpool(bsz, d, ldp, lqp, info.num_cores,
                          info.num_subcores, chunk=32)
    maxq, maxd = sc(table_n, wm)
    return _cosine(maxq, maxd)
